# baseline (speedup 1.0000x reference)
import numpy as np
import jax
import jax.numpy as jnp
from jax.experimental import pallas as pl

DEPTH = 1000
ROWS = 4096 * 26
W = 1024
R_BLK = 1000
J_BLK = 1024
GRID = ROWS // J_BLK

_a = (np.arange(R_BLK) * W) // DEPTH
_PA = np.zeros((R_BLK, J_BLK), np.float32)
_PA[np.arange(R_BLK), _a] = 1.0
_PB = np.zeros((R_BLK, J_BLK), np.float32)
_PB[np.arange(R_BLK), _a + 1] = 1.0
_PC = np.zeros((R_BLK, J_BLK), np.float32)
_c_ok = _a + 2 < J_BLK
_PC[np.arange(R_BLK)[_c_ok], (_a + 2)[_c_ok]] = 1.0


def _body(idx_ref, pa_ref, pb_ref, pc_ref, out_ref):
    j_iota = jax.lax.broadcasted_iota(jnp.int32, (1, J_BLK), 1)
    lt_row = (j_iota * DEPTH + idx_ref[0]).astype(jnp.float32)
    lt_col = lt_row.reshape(J_BLK, 1)
    t_a = jnp.dot(pa_ref[...], lt_col, preferred_element_type=jnp.float32,
                  precision=jax.lax.Precision.HIGHEST)
    t_b = jnp.dot(pb_ref[...], lt_col, preferred_element_type=jnp.float32,
                  precision=jax.lax.Precision.HIGHEST)
    t_c = jnp.dot(pc_ref[...], lt_col, preferred_element_type=jnp.float32,
                  precision=jax.lax.Precision.HIGHEST)
    rowoff = jax.lax.broadcasted_iota(jnp.int32, (R_BLK, 1), 0) * W
    c_a = t_a.astype(jnp.int32) - rowoff
    c_b = t_b.astype(jnp.int32) - rowoff
    c_c = t_c.astype(jnp.int32) - rowoff
    lane = jax.lax.broadcasted_iota(jnp.int32, (R_BLK, W), 1)
    out_ref[...] = ((lane == c_a) | (lane == c_b) | (lane == c_c)).astype(
        jnp.float32)


def kernel(inputs):
    idx = inputs.astype(jnp.int32).reshape(GRID, 1, J_BLK)
    out = pl.pallas_call(
        _body,
        grid=(GRID,),
        in_specs=[
            pl.BlockSpec((1, 1, J_BLK), lambda i: (i, 0, 0)),
            pl.BlockSpec((R_BLK, J_BLK), lambda i: (0, 0)),
            pl.BlockSpec((R_BLK, J_BLK), lambda i: (0, 0)),
            pl.BlockSpec((R_BLK, J_BLK), lambda i: (0, 0)),
        ],
        out_specs=pl.BlockSpec((R_BLK, W), lambda i: (i, 0)),
        out_shape=jax.ShapeDtypeStruct((ROWS * DEPTH // W, W), jnp.float32),
    )(idx, jnp.asarray(_PA), jnp.asarray(_PB), jnp.asarray(_PC))
    return out.reshape(4096, 26, DEPTH)


# speedup vs baseline: 20.4629x; 20.4629x over previous
import jax
import jax.numpy as jnp
from jax.experimental import pallas as pl

DEPTH = 1000
N0 = 4096
N1 = 26


def _body(idx_ref, out_ref):
    row = idx_ref[0]
    iota = jax.lax.broadcasted_iota(jnp.int32, (DEPTH, N0), 0)
    out_ref[...] = (iota == row).astype(jnp.float32)


def kernel(inputs):
    idx_t = inputs.astype(jnp.int32).T.reshape(N1, 1, N0)
    out_t = pl.pallas_call(
        _body,
        grid=(N1,),
        in_specs=[pl.BlockSpec((1, 1, N0), lambda i: (i, 0, 0))],
        out_specs=pl.BlockSpec((DEPTH, N0), lambda i: (i, 0)),
        out_shape=jax.ShapeDtypeStruct((N1 * DEPTH, N0), jnp.float32),
    )(idx_t)
    return out_t.reshape(N1, DEPTH, N0).transpose(2, 0, 1)
